# baseline (speedup 1.0000x reference)
import functools

import numpy as np
import jax
import jax.numpy as jnp
from jax import lax
from jax.experimental import pallas as pl
from jax.experimental.pallas import tpu as pltpu
from jax.experimental.pallas import tpu_sc as plsc

_NC = 2
_NS = 16
_NW = _NC * _NS
_L = 16


def _positional_encoding(length: int, d_model: int) -> np.ndarray:
    positions = np.arange(length)[:, None]
    dims = np.arange(d_model)[None, :]
    angle_rates = 1.0 / np.power(10000.0, 2 * (dims // 2) / np.float32(d_model))
    angle_rads = positions * angle_rates
    pos = np.zeros((length, d_model), dtype=np.float32)
    pos[:, 0::2] = np.sin(angle_rads[:, 0::2])
    pos[:, 1::2] = np.cos(angle_rads[:, 1::2])
    return pos


def _make_sc_kernel(B: int, S: int, D: int):
    total_rows = B * S
    rows_per_w = total_rows // _NW
    chunks_per_w = rows_per_w // S
    half = S // 2
    scale = float(np.sqrt(np.float32(D)))
    groups = D // _L

    mesh = plsc.VectorSubcoreMesh(core_axis_name="c", subcore_axis_name="s")

    @functools.partial(
        pl.kernel,
        mesh=mesh,
        out_type=jax.ShapeDtypeStruct((total_rows, D), jnp.float32),
        compiler_params=pltpu.CompilerParams(use_tc_tiling_on_sc=False),
        scratch_types=[
            pltpu.VMEM((2, half), jnp.int32),
            pltpu.VMEM((S, D), jnp.float32),
            pltpu.VMEM((S, D), jnp.float32),
            pltpu.SemaphoreType.DMA,
        ],
    )
    def k(x_hbm, table_hbm, pos_hbm, out_hbm, idx_v, buf_v, pos_v, sem):
        wid = lax.axis_index("s") * _NC + lax.axis_index("c")
        pltpu.sync_copy(pos_hbm, pos_v)
        base_chunk = wid * chunks_per_w

        def chunk_body(i, carry):
            c = base_chunk + i
            r0 = c * S
            pltpu.sync_copy(x_hbm.at[pl.ds(2 * c, 2)], idx_v)
            cp0 = pltpu.async_copy(
                table_hbm.at[idx_v.at[0]], buf_v.at[pl.ds(0, half)], sem)
            cp1 = pltpu.async_copy(
                table_hbm.at[idx_v.at[1]], buf_v.at[pl.ds(half, half)], sem)
            cp0.wait()
            cp1.wait()

            def row_body(r, carry2):
                for g in range(groups):
                    sl = pl.ds(g * _L, _L)
                    buf_v[r, sl] = buf_v[r, sl] * scale + pos_v[r, sl]
                return carry2

            lax.fori_loop(0, S, row_body, 0, unroll=4)
            pltpu.sync_copy(buf_v, out_hbm.at[pl.ds(r0, S)])
            return carry

        lax.fori_loop(0, chunks_per_w, chunk_body, 0)

    return k


def kernel(x, table):
    B, S = x.shape
    V, D = table.shape
    pos = jnp.asarray(_positional_encoding(S, D))
    x_flat = x.reshape(B * S // (S // 2), S // 2).astype(jnp.int32)
    k = _make_sc_kernel(B, S, D)
    out = k(x_flat, table, pos)
    return out.reshape(B, S, D)


# speedup vs baseline: 1.1480x; 1.1480x over previous
import functools

import numpy as np
import jax
import jax.numpy as jnp
from jax import lax
from jax.experimental import pallas as pl
from jax.experimental.pallas import tpu as pltpu
from jax.experimental.pallas import tpu_sc as plsc

_NC = 2
_NS = 16
_NW = _NC * _NS
_L = 16


def _positional_encoding(length: int, d_model: int) -> np.ndarray:
    positions = np.arange(length)[:, None]
    dims = np.arange(d_model)[None, :]
    angle_rates = 1.0 / np.power(10000.0, 2 * (dims // 2) / np.float32(d_model))
    angle_rads = positions * angle_rates
    pos = np.zeros((length, d_model), dtype=np.float32)
    pos[:, 0::2] = np.sin(angle_rads[:, 0::2])
    pos[:, 1::2] = np.cos(angle_rads[:, 1::2])
    return pos


def _make_sc_kernel(B: int, S: int, D: int, DP: int):
    rows_per_w = B // _NW
    half = S // 2
    scale = float(np.sqrt(np.float32(D)))
    groups = D // _L

    mesh = plsc.VectorSubcoreMesh(core_axis_name="c", subcore_axis_name="s")

    @functools.partial(
        pl.kernel,
        mesh=mesh,
        out_type=jax.ShapeDtypeStruct((B, S, D), jnp.float32),
        compiler_params=pltpu.CompilerParams(use_tc_tiling_on_sc=False),
        scratch_types=[
            pltpu.VMEM((2 * rows_per_w, half), jnp.int32),
            pltpu.VMEM((S, DP), jnp.float32),
            pltpu.VMEM((S, DP), jnp.float32),
            pltpu.VMEM((S, D), jnp.float32),
            pltpu.VMEM((S, D), jnp.float32),
            pltpu.VMEM((S, D), jnp.float32),
            pltpu.SemaphoreType.DMA,
            pltpu.SemaphoreType.DMA,
            pltpu.SemaphoreType.DMA,
            pltpu.SemaphoreType.DMA,
        ],
    )
    def k(x_hbm, table_hbm, pos_hbm, out_hbm,
          idx_v, buf0, buf1, ob0, ob1, pos_v, gs0, gs1, os0, os1):
        wid = lax.axis_index("s") * _NC + lax.axis_index("c")
        bufs = (buf0, buf1)
        obufs = (ob0, ob1)
        gsems = (gs0, gs1)
        osems = (os0, os1)

        pltpu.sync_copy(pos_hbm, pos_v)
        pltpu.sync_copy(x_hbm.at[wid], idx_v)
        base_b = wid * rows_per_w

        def gather_chunk(i, slot):
            pltpu.async_copy(
                table_hbm.at[idx_v.at[2 * i]],
                bufs[slot].at[pl.ds(0, half)], gsems[slot])
            pltpu.async_copy(
                table_hbm.at[idx_v.at[2 * i + 1]],
                bufs[slot].at[pl.ds(half, half)], gsems[slot])

        def wait_gather(i, slot):
            pltpu.make_async_copy(
                table_hbm.at[idx_v.at[2 * i]],
                bufs[slot].at[pl.ds(0, half)], gsems[slot]).wait()
            pltpu.make_async_copy(
                table_hbm.at[idx_v.at[2 * i + 1]],
                bufs[slot].at[pl.ds(half, half)], gsems[slot]).wait()

        def wait_out(i, slot):
            pltpu.make_async_copy(
                obufs[slot], out_hbm.at[base_b + i], osems[slot]).wait()

        gather_chunk(0, 0)
        gather_chunk(1, 1)

        def step(i, slot):
            buf = bufs[slot]
            obuf = obufs[slot]
            wait_gather(i, slot)

            @pl.when(i >= 2)
            def _():
                wait_out(i - 2, slot)

            def row_body(r, carry):
                for g in range(groups):
                    sl = pl.ds(g * _L, _L)
                    obuf[r, sl] = buf[r, sl] * scale + pos_v[r, sl]
                return carry

            lax.fori_loop(0, S, row_body, 0, unroll=8)
            pltpu.async_copy(obuf, out_hbm.at[base_b + i], osems[slot])

            @pl.when(i + 2 < rows_per_w)
            def _():
                gather_chunk(i + 2, slot)

        def pair_body(j, carry):
            step(2 * j, 0)
            step(2 * j + 1, 1)
            return carry

        lax.fori_loop(0, rows_per_w // 2, pair_body, 0)
        wait_out(rows_per_w - 2, 0)
        wait_out(rows_per_w - 1, 1)

    return k


def kernel(x, table):
    B, S = x.shape
    V, D = table.shape
    DP = 2 * D
    pos = jnp.asarray(_positional_encoding(S, D))
    table_p = jnp.pad(table, ((0, 0), (0, DP - D)))
    x3 = x.reshape(_NW, (B // _NW) * 2, S // 2).astype(jnp.int32)
    k = _make_sc_kernel(B, S, D, DP)
    return k(x3, table_p, pos)
